# baseline (speedup 1.0000x reference)
import functools

import jax
import jax.numpy as jnp
from jax import lax
from jax.experimental import pallas as pl
from jax.experimental.pallas import tpu as pltpu
from jax.experimental.pallas import tpu_sc as plsc

_MM_DIMS_CONTRACT0 = (((0,), (0,)), ((), ()))
_MM_DIMS_ROWXCOL = (((1,), (0,)), ((), ()))


def _bf16_mm(a, b, dims):
    return jax.lax.dot_general(a, b, dims,
                               preferred_element_type=jnp.float32,
                               precision=jax.lax.Precision.DEFAULT)


def _trunc16(v):
    bits = jax.lax.bitcast_convert_type(v, jnp.uint32)
    return jax.lax.bitcast_convert_type(bits & jnp.uint32(0xFFFF0000),
                                        jnp.float32)


def _vq_tc_kernel(z_ref, dic_ref, idx_ref, idxf_ref, dict_ref):
    dic = dic_ref[...]
    c_norm = jnp.sum(dic * dic, axis=0)
    B = z_ref.shape[0]
    F, K = dic.shape
    H, W = z_ref.shape[2], z_ref.shape[3]
    HW = H * W
    N = B * HW

    dic_h32 = _trunc16(dic)
    dic_l32 = dic - dic_h32
    ndic_h = (-2.0 * dic_h32).astype(jnp.bfloat16)
    ndic_l = (-2.0 * dic_l32).astype(jnp.bfloat16)
    dic_h = dic_h32.astype(jnp.bfloat16)
    dic_m32 = _trunc16(dic_l32)
    dic_m = dic_m32.astype(jnp.bfloat16)
    dic_t = (dic_l32 - dic_m32).astype(jnp.bfloat16)

    cn_h32 = _trunc16(c_norm)
    cn_h = cn_h32.astype(jnp.bfloat16)
    cn_l = (c_norm - cn_h32).astype(jnp.bfloat16)

    x = jnp.concatenate(
        [z_ref[b].reshape(F, HW) for b in range(B)], axis=1)
    x_h32 = _trunc16(x)
    x_h = x_h32.astype(jnp.bfloat16)
    x_l = (x - x_h32).astype(jnp.bfloat16)

    ones_row = jnp.ones((1, N), jnp.bfloat16)
    lhs = jnp.concatenate(
        [ndic_h, ndic_h, ndic_l, cn_h[None, :], cn_l[None, :]], axis=0)
    rhs = jnp.concatenate([x_h, x_l, x_h, ones_row, ones_row], axis=0)
    dist = _bf16_mm(lhs, rhs, _MM_DIMS_CONTRACT0)
    iota_k = jax.lax.broadcasted_iota(jnp.int32, (K, N), 0)
    i1 = jnp.argmin(dist, axis=0).astype(jnp.int32)
    eq1 = iota_k == i1[None, :]
    masked = jnp.where(eq1, jnp.inf, dist)
    i2 = jnp.argmin(masked, axis=0).astype(jnp.int32)

    ohb = jnp.concatenate(
        [eq1, iota_k == i2[None, :]], axis=1).astype(jnp.bfloat16)
    c12 = ((_bf16_mm(dic_h, ohb, _MM_DIMS_ROWXCOL)
            + _bf16_mm(dic_m, ohb, _MM_DIMS_ROWXCOL))
           + _bf16_mm(dic_t, ohb, _MM_DIMS_ROWXCOL))
    c1 = c12[:, :N]
    c2 = c12[:, N:]

    e1 = jnp.zeros((1, N), jnp.float32)
    e2 = jnp.zeros((1, N), jnp.float32)
    for f in range(F):
        d1f = x[f:f + 1, :] - c1[f:f + 1, :]
        d2f = x[f:f + 1, :] - c2[f:f + 1, :]
        e1 = e1 + d1f * d1f
        e2 = e2 + d2f * d2f

    take2 = (e2 < e1) | ((e2 == e1) & (i2[None, :] < i1[None, :]))
    idx = jnp.where(take2[0], i2, i1)
    idxf_ref[...] = idx[None, :]
    dict_ref[...] = jnp.pad(dic.T, ((0, 0), (0, 128 - F)))
    idx2 = idx[None, :]
    for b in range(B):
        for h in range(H):
            idx_ref[b, h, :] = idx2[0, b * HW + h * W: b * HW + (h + 1) * W]


def _make_sc_gather(V, D, B):
    info = plsc.get_sparse_core_info()
    NC, NS, L = info.num_cores, info.num_subcores, info.num_lanes
    NW = NC * NS
    assert D % L == 0 and B % (8 * NW) == 0
    b_per_w = B // NW
    mesh = plsc.VectorSubcoreMesh(core_axis_name="c", subcore_axis_name="s")

    @functools.partial(
        pl.kernel, mesh=mesh,
        out_type=jax.ShapeDtypeStruct((B, D), jnp.float32),
        scratch_types=[
            pltpu.VMEM((b_per_w,), jnp.int32),
            pltpu.VMEM((b_per_w, D), jnp.float32),
            pltpu.SemaphoreType.DMA,
        ],
    )
    def k(table_hbm, idx_hbm, out_hbm, idx_v, rows_v, sem):
        wid = lax.axis_index("s") * NC + lax.axis_index("c")
        base = wid * b_per_w
        pltpu.sync_copy(idx_hbm.at[pl.ds(base, b_per_w)], idx_v)
        pltpu.async_copy(table_hbm.at[idx_v], rows_v, sem).wait()
        pltpu.sync_copy(rows_v, out_hbm.at[pl.ds(base, b_per_w)])

    return k


@functools.partial(jax.jit, static_argnames=())
def kernel(z, z_dic):
    B, F, H, W = z.shape
    _F, K = z_dic.shape
    N = B * H * W
    idx, idxf, dic_t = pl.pallas_call(
        _vq_tc_kernel,
        out_shape=(
            jax.ShapeDtypeStruct((B, H, W), jnp.int32),
            jax.ShapeDtypeStruct((1, N), jnp.int32),
            jax.ShapeDtypeStruct((K, 128), jnp.float32),
        ),
    )(z, z_dic)
    rows = _make_sc_gather(K, 128, N)(dic_t, idxf.reshape(N))
    zq = jnp.transpose(rows.reshape(B, H, W, 128)[..., :F], (0, 3, 1, 2))
    return (zq, idx)


# speedup vs baseline: 3.3734x; 3.3734x over previous
import functools

import jax
import jax.numpy as jnp
from jax.experimental import pallas as pl

_MM_DIMS_CONTRACT0 = (((0,), (0,)), ((), ()))
_MM_DIMS_ROWXCOL = (((1,), (0,)), ((), ()))


def _bf16_mm(a, b, dims):
    return jax.lax.dot_general(a, b, dims,
                               preferred_element_type=jnp.float32,
                               precision=jax.lax.Precision.DEFAULT)


def _trunc16(v):
    bits = jax.lax.bitcast_convert_type(v, jnp.uint32)
    return jax.lax.bitcast_convert_type(bits & jnp.uint32(0xFFFF0000),
                                        jnp.float32)


def _vq_kernel(z_ref, dic_ref, zq_ref, idx_ref):
    dic = dic_ref[...]
    c_norm = jnp.sum(dic * dic, axis=0)
    B = z_ref.shape[0]
    F, K = dic.shape
    H, W = z_ref.shape[2], z_ref.shape[3]
    HW = H * W
    N = B * HW

    dic_h32 = _trunc16(dic)
    dic_l32 = dic - dic_h32
    ndic_h = (-2.0 * dic_h32).astype(jnp.bfloat16)
    ndic_l = (-2.0 * dic_l32).astype(jnp.bfloat16)
    dic_h = dic_h32.astype(jnp.bfloat16)
    dic_m32 = _trunc16(dic_l32)
    dic_m = dic_m32.astype(jnp.bfloat16)
    dic_t = (dic_l32 - dic_m32).astype(jnp.bfloat16)

    cn_h32 = _trunc16(c_norm)
    cn_h = cn_h32.astype(jnp.bfloat16)
    cn_l = (c_norm - cn_h32).astype(jnp.bfloat16)

    x = jnp.concatenate(
        [z_ref[b].reshape(F, HW) for b in range(B)], axis=1)
    x_h32 = _trunc16(x)
    x_h = x_h32.astype(jnp.bfloat16)
    x_l = (x - x_h32).astype(jnp.bfloat16)

    ones_row = jnp.ones((1, N), jnp.bfloat16)
    lhs = jnp.concatenate(
        [ndic_h, ndic_h, ndic_l, cn_h[None, :], cn_l[None, :]], axis=0)
    rhs = jnp.concatenate([x_h, x_l, x_h, ones_row, ones_row], axis=0)
    dist = _bf16_mm(lhs, rhs, _MM_DIMS_CONTRACT0)
    iota_k = jax.lax.broadcasted_iota(jnp.int32, (K, N), 0)
    i1 = jnp.argmin(dist, axis=0).astype(jnp.int32)
    eq1 = iota_k == i1[None, :]
    masked = jnp.where(eq1, jnp.inf, dist)
    i2 = jnp.argmin(masked, axis=0).astype(jnp.int32)

    ohb = jnp.concatenate(
        [eq1, iota_k == i2[None, :]], axis=1).astype(jnp.bfloat16)
    c12 = ((_bf16_mm(dic_h, ohb, _MM_DIMS_ROWXCOL)
            + _bf16_mm(dic_m, ohb, _MM_DIMS_ROWXCOL))
           + _bf16_mm(dic_t, ohb, _MM_DIMS_ROWXCOL))
    c1 = c12[:, :N]
    c2 = c12[:, N:]

    e1 = jnp.zeros((1, N), jnp.float32)
    e2 = jnp.zeros((1, N), jnp.float32)
    for f in range(F):
        d1f = x[f:f + 1, :] - c1[f:f + 1, :]
        d2f = x[f:f + 1, :] - c2[f:f + 1, :]
        e1 = e1 + d1f * d1f
        e2 = e2 + d2f * d2f

    take2 = (e2 < e1) | ((e2 == e1) & (i2[None, :] < i1[None, :]))
    idx = jnp.where(take2[0], i2, i1)
    zq = jnp.where(take2, c2, c1)
    idx2 = idx[None, :]
    for b in range(B):
        zq_ref[b] = zq[:, b * HW:(b + 1) * HW].reshape(F, H, W)
        for h in range(H):
            idx_ref[b, h, :] = idx2[0, b * HW + h * W: b * HW + (h + 1) * W]


@functools.partial(jax.jit, static_argnames=())
def kernel(z, z_dic):
    B, F, H, W = z.shape
    zq, idx = pl.pallas_call(
        _vq_kernel,
        out_shape=(
            jax.ShapeDtypeStruct((B, F, H, W), jnp.float32),
            jax.ShapeDtypeStruct((B, H, W), jnp.int32),
        ),
    )(z, z_dic)
    return (zq, idx)
